# baseline (speedup 1.0000x reference)
import jax
import jax.numpy as jnp
from jax import lax
from jax.experimental import pallas as pl
from jax.experimental.pallas import tpu as pltpu
from jax.experimental.pallas import tpu_sc as plsc

NC = 2
NS = 16
L = 16
NW = NC * NS

CHUNK = 128


def _body(nodes_hbm, w_hbm, h_hbm, wb_hbm, hb_hbm, out_hbm,
          idx_v, w_rows, h_rows, wb_v, hb_v, out_v, sem):
    n_chunks = nodes_hbm.shape[0] // NW
    wid = lax.axis_index("s") * NC + lax.axis_index("c")
    base_row = wid * n_chunks

    pltpu.sync_copy(nodes_hbm.at[pl.ds(base_row, n_chunks)], idx_v)

    for k in range(n_chunks):
        idx_k = idx_v.at[k]
        cw = pltpu.async_copy(w_hbm.at[idx_k], w_rows, sem)
        ch = pltpu.async_copy(h_hbm.at[idx_k], h_rows, sem)
        cwb = pltpu.async_copy(wb_hbm.at[idx_k], wb_v, sem)
        chb = pltpu.async_copy(hb_hbm.at[idx_k], hb_v, sem)
        cw.wait()
        ch.wait()
        cwb.wait()
        chb.wait()

        def group(g, _):
            rows = g * L + lax.iota(jnp.int32, L)
            acc0 = wb_v[pl.ds(g * L, L)] + hb_v[pl.ds(g * L, L)]
            acc1 = jnp.zeros((L,), jnp.float32)
            for f in range(0, 32, 2):
                col0 = jnp.full((L,), f, jnp.int32)
                col1 = jnp.full((L,), f + 1, jnp.int32)
                acc0 += (plsc.load_gather(w_rows, [rows, col0])
                         * plsc.load_gather(h_rows, [rows, col0]))
                acc1 += (plsc.load_gather(w_rows, [rows, col1])
                         * plsc.load_gather(h_rows, [rows, col1]))
            out_v[pl.ds(g * L, L)] = acc0 + acc1
            return 0

        lax.fori_loop(0, CHUNK // L, group, 0)
        pltpu.sync_copy(out_v,
                        out_hbm.at[pl.ds((base_row + k) * CHUNK, CHUNK)])


def kernel(nodes, W, H, w_bias, h_bias):
    batch = nodes.shape[0]
    nodes2d = jnp.reshape(nodes.astype(jnp.int32), (batch // CHUNK, CHUNK))
    wb = jnp.reshape(w_bias, (w_bias.shape[0],))
    hb = jnp.reshape(h_bias, (h_bias.shape[0],))

    mesh = plsc.VectorSubcoreMesh(core_axis_name="c", subcore_axis_name="s")
    n_chunks = nodes2d.shape[0] // NW
    run = pl.kernel(
        _body,
        out_type=jax.ShapeDtypeStruct((batch,), jnp.float32),
        mesh=mesh,
        scratch_types=[
            pltpu.VMEM((n_chunks, CHUNK), jnp.int32),
            pltpu.VMEM((CHUNK, 32), jnp.float32),
            pltpu.VMEM((CHUNK, 32), jnp.float32),
            pltpu.VMEM((CHUNK,), jnp.float32),
            pltpu.VMEM((CHUNK,), jnp.float32),
            pltpu.VMEM((CHUNK,), jnp.float32),
            pltpu.SemaphoreType.DMA,
        ],
        compiler_params=pltpu.CompilerParams(
            needs_layout_passes=False, use_tc_tiling_on_sc=False),
    )
    return run(nodes2d, W, H, wb, hb)


# speedup vs baseline: 2.2996x; 2.2996x over previous
import jax
import jax.numpy as jnp
from jax import lax
from jax.experimental import pallas as pl
from jax.experimental.pallas import tpu as pltpu
from jax.experimental.pallas import tpu_sc as plsc

NC = 2
NS = 16
NW = NC * NS

CHUNK = 128
NF = 32
BLK = 2048


def _dense_body(wt_ref, ht_ref, wb_ref, hb_ref, out_ref):
    prod = wt_ref[...] * ht_ref[...]
    out_ref[...] = jnp.sum(prod, axis=0) + wb_ref[...] + hb_ref[...]


def _gather_body(nodes_hbm, dsum_hbm, out_hbm, idx_v, val_v, sem):
    n_chunks = nodes_hbm.shape[0] // NW
    wid = lax.axis_index("s") * NC + lax.axis_index("c")
    base_row = wid * n_chunks

    pltpu.sync_copy(nodes_hbm.at[pl.ds(base_row, n_chunks)], idx_v)
    for k in range(n_chunks):
        pltpu.async_copy(dsum_hbm.at[idx_v.at[k]], val_v.at[k], sem)
    for k in range(n_chunks):
        pltpu.make_async_copy(dsum_hbm.at[idx_v.at[k]], val_v.at[k],
                              sem).wait()
    for k in range(n_chunks):
        pltpu.sync_copy(
            val_v.at[k],
            out_hbm.at[pl.ds((base_row + k) * CHUNK, CHUNK)])


def kernel(nodes, W, H, w_bias, h_bias):
    batch = nodes.shape[0]
    nn = W.shape[0]
    wt = jnp.transpose(W)
    ht = jnp.transpose(H)
    wb = jnp.reshape(w_bias, (nn,))
    hb = jnp.reshape(h_bias, (nn,))

    grid = (nn + BLK - 1) // BLK
    dsum = pl.pallas_call(
        _dense_body,
        grid=(grid,),
        in_specs=[
            pl.BlockSpec((NF, BLK), lambda i: (0, i)),
            pl.BlockSpec((NF, BLK), lambda i: (0, i)),
            pl.BlockSpec((BLK,), lambda i: (i,)),
            pl.BlockSpec((BLK,), lambda i: (i,)),
        ],
        out_specs=pl.BlockSpec((BLK,), lambda i: (i,)),
        out_shape=jax.ShapeDtypeStruct((nn,), jnp.float32),
        compiler_params=pltpu.CompilerParams(
            dimension_semantics=("arbitrary",)),
    )(wt, ht, wb, hb)

    nodes2d = jnp.reshape(nodes.astype(jnp.int32), (batch // CHUNK, CHUNK))
    mesh = plsc.VectorSubcoreMesh(core_axis_name="c", subcore_axis_name="s")
    n_chunks = nodes2d.shape[0] // NW
    run = pl.kernel(
        _gather_body,
        out_type=jax.ShapeDtypeStruct((batch,), jnp.float32),
        mesh=mesh,
        scratch_types=[
            pltpu.VMEM((n_chunks, CHUNK), jnp.int32),
            pltpu.VMEM((n_chunks, CHUNK), jnp.float32),
            pltpu.SemaphoreType.DMA,
        ],
    )
    return run(nodes2d, dsum)


# speedup vs baseline: 4.0411x; 1.7573x over previous
import jax
import jax.numpy as jnp
from jax import lax
from jax.experimental import pallas as pl
from jax.experimental.pallas import tpu as pltpu
from jax.experimental.pallas import tpu_sc as plsc

NC = 2
NS = 16
NW = NC * NS

CHUNK = 128
NF = 32
BLK = 8192


def _dense_body(wt_ref, ht_ref, wb_ref, hb_ref, out_ref):
    prod = wt_ref[...] * ht_ref[...]
    out_ref[...] = jnp.sum(prod, axis=0) + wb_ref[...] + hb_ref[...]


def _gather_body(nodes_hbm, dsum_hbm, out_hbm, idx_v, val_v, sem):
    n_chunks = nodes_hbm.shape[0] // NW
    wid = lax.axis_index("s") * NC + lax.axis_index("c")
    base_row = wid * n_chunks

    pltpu.sync_copy(nodes_hbm.at[pl.ds(base_row, n_chunks)], idx_v)
    for k in range(n_chunks):
        pltpu.async_copy(dsum_hbm.at[idx_v.at[k]], val_v.at[k], sem)
    for k in range(n_chunks):
        pltpu.make_async_copy(dsum_hbm.at[idx_v.at[k]], val_v.at[k],
                              sem).wait()
    for k in range(n_chunks):
        pltpu.sync_copy(
            val_v.at[k],
            out_hbm.at[pl.ds((base_row + k) * CHUNK, CHUNK)])


def kernel(nodes, W, H, w_bias, h_bias):
    batch = nodes.shape[0]
    nn = W.shape[0]
    wt = jnp.transpose(W)
    ht = jnp.transpose(H)
    wb = jnp.reshape(w_bias, (nn,))
    hb = jnp.reshape(h_bias, (nn,))

    grid = (nn + BLK - 1) // BLK
    dsum = pl.pallas_call(
        _dense_body,
        grid=(grid,),
        in_specs=[
            pl.BlockSpec((NF, BLK), lambda i: (0, i)),
            pl.BlockSpec((NF, BLK), lambda i: (0, i)),
            pl.BlockSpec((BLK,), lambda i: (i,)),
            pl.BlockSpec((BLK,), lambda i: (i,)),
        ],
        out_specs=pl.BlockSpec((BLK,), lambda i: (i,)),
        out_shape=jax.ShapeDtypeStruct((nn,), jnp.float32),
        compiler_params=pltpu.CompilerParams(
            dimension_semantics=("arbitrary",)),
    )(wt, ht, wb, hb)

    nodes2d = jnp.reshape(nodes.astype(jnp.int32), (batch // CHUNK, CHUNK))
    mesh = plsc.VectorSubcoreMesh(core_axis_name="c", subcore_axis_name="s")
    n_chunks = nodes2d.shape[0] // NW
    run = pl.kernel(
        _gather_body,
        out_type=jax.ShapeDtypeStruct((batch,), jnp.float32),
        mesh=mesh,
        scratch_types=[
            pltpu.VMEM((n_chunks, CHUNK), jnp.int32),
            pltpu.VMEM((n_chunks, CHUNK), jnp.float32),
            pltpu.SemaphoreType.DMA,
        ],
    )
    return run(nodes2d, dsum)


# speedup vs baseline: 4.8713x; 1.2054x over previous
import jax
import jax.numpy as jnp
from jax import lax
from jax.experimental import pallas as pl
from jax.experimental.pallas import tpu as pltpu
from jax.experimental.pallas import tpu_sc as plsc

NC = 2
NS = 16
NW = NC * NS

CHUNK = 128
NF = 32
BLK = 32768


def _dense_body(wt_ref, ht_ref, wb_ref, hb_ref, out_ref):
    prod = wt_ref[...] * ht_ref[...]
    out_ref[...] = jnp.sum(prod, axis=0) + wb_ref[...] + hb_ref[...]


def _gather_body(nodes_hbm, dsum_hbm, out_hbm, idx_v, val_v, sem):
    n_chunks = nodes_hbm.shape[0] // NW
    wid = lax.axis_index("s") * NC + lax.axis_index("c")
    base_row = wid * n_chunks

    pltpu.sync_copy(nodes_hbm.at[pl.ds(base_row, n_chunks)], idx_v)
    for k in range(n_chunks):
        pltpu.async_copy(dsum_hbm.at[idx_v.at[k]], val_v.at[k], sem)
    for k in range(n_chunks):
        pltpu.make_async_copy(dsum_hbm.at[idx_v.at[k]], val_v.at[k],
                              sem).wait()
    for k in range(n_chunks):
        pltpu.sync_copy(
            val_v.at[k],
            out_hbm.at[pl.ds((base_row + k) * CHUNK, CHUNK)])


def kernel(nodes, W, H, w_bias, h_bias):
    batch = nodes.shape[0]
    nn = W.shape[0]
    wt = jnp.transpose(W)
    ht = jnp.transpose(H)
    wb = jnp.reshape(w_bias, (nn,))
    hb = jnp.reshape(h_bias, (nn,))

    grid = (nn + BLK - 1) // BLK
    dsum = pl.pallas_call(
        _dense_body,
        grid=(grid,),
        in_specs=[
            pl.BlockSpec((NF, BLK), lambda i: (0, i)),
            pl.BlockSpec((NF, BLK), lambda i: (0, i)),
            pl.BlockSpec((BLK,), lambda i: (i,)),
            pl.BlockSpec((BLK,), lambda i: (i,)),
        ],
        out_specs=pl.BlockSpec((BLK,), lambda i: (i,)),
        out_shape=jax.ShapeDtypeStruct((nn,), jnp.float32),
        compiler_params=pltpu.CompilerParams(
            dimension_semantics=("arbitrary",)),
    )(wt, ht, wb, hb)

    nodes2d = jnp.reshape(nodes.astype(jnp.int32), (batch // CHUNK, CHUNK))
    mesh = plsc.VectorSubcoreMesh(core_axis_name="c", subcore_axis_name="s")
    n_chunks = nodes2d.shape[0] // NW
    run = pl.kernel(
        _gather_body,
        out_type=jax.ShapeDtypeStruct((batch,), jnp.float32),
        mesh=mesh,
        scratch_types=[
            pltpu.VMEM((n_chunks, CHUNK), jnp.int32),
            pltpu.VMEM((n_chunks, CHUNK), jnp.float32),
            pltpu.SemaphoreType.DMA,
        ],
    )
    return run(nodes2d, dsum)


# speedup vs baseline: 4.8751x; 1.0008x over previous
import jax
import jax.numpy as jnp
from jax import lax
from jax.experimental import pallas as pl
from jax.experimental.pallas import tpu as pltpu
from jax.experimental.pallas import tpu_sc as plsc

NC = 2
NS = 16
NW = NC * NS

CHUNK = 128
NF = 32
BLK = 65536


def _dense_body(wt_ref, ht_ref, wb_ref, hb_ref, out_ref):
    prod = wt_ref[...] * ht_ref[...]
    out_ref[...] = jnp.sum(prod, axis=0) + wb_ref[...] + hb_ref[...]


def _gather_body(nodes_hbm, dsum_hbm, out_hbm, idx_v, val_v, sem):
    n_chunks = nodes_hbm.shape[0] // NW
    wid = lax.axis_index("s") * NC + lax.axis_index("c")
    base_row = wid * n_chunks

    pltpu.sync_copy(nodes_hbm.at[pl.ds(base_row, n_chunks)], idx_v)
    for k in range(n_chunks):
        pltpu.async_copy(dsum_hbm.at[idx_v.at[k]], val_v.at[k], sem)
    for k in range(n_chunks):
        pltpu.make_async_copy(dsum_hbm.at[idx_v.at[k]], val_v.at[k],
                              sem).wait()
    for k in range(n_chunks):
        pltpu.sync_copy(
            val_v.at[k],
            out_hbm.at[pl.ds((base_row + k) * CHUNK, CHUNK)])


def kernel(nodes, W, H, w_bias, h_bias):
    batch = nodes.shape[0]
    nn = W.shape[0]
    wt = jnp.transpose(W)
    ht = jnp.transpose(H)
    wb = jnp.reshape(w_bias, (nn,))
    hb = jnp.reshape(h_bias, (nn,))

    grid = (nn + BLK - 1) // BLK
    dsum = pl.pallas_call(
        _dense_body,
        grid=(grid,),
        in_specs=[
            pl.BlockSpec((NF, BLK), lambda i: (0, i)),
            pl.BlockSpec((NF, BLK), lambda i: (0, i)),
            pl.BlockSpec((BLK,), lambda i: (i,)),
            pl.BlockSpec((BLK,), lambda i: (i,)),
        ],
        out_specs=pl.BlockSpec((BLK,), lambda i: (i,)),
        out_shape=jax.ShapeDtypeStruct((nn,), jnp.float32),
        compiler_params=pltpu.CompilerParams(
            dimension_semantics=("arbitrary",)),
    )(wt, ht, wb, hb)

    nodes2d = jnp.reshape(nodes.astype(jnp.int32), (batch // CHUNK, CHUNK))
    mesh = plsc.VectorSubcoreMesh(core_axis_name="c", subcore_axis_name="s")
    n_chunks = nodes2d.shape[0] // NW
    run = pl.kernel(
        _gather_body,
        out_type=jax.ShapeDtypeStruct((batch,), jnp.float32),
        mesh=mesh,
        scratch_types=[
            pltpu.VMEM((n_chunks, CHUNK), jnp.int32),
            pltpu.VMEM((n_chunks, CHUNK), jnp.float32),
            pltpu.SemaphoreType.DMA,
        ],
    )
    return run(nodes2d, dsum)


# speedup vs baseline: 4.9203x; 1.0093x over previous
import jax
import jax.numpy as jnp
from jax import lax
from jax.experimental import pallas as pl
from jax.experimental.pallas import tpu as pltpu
from jax.experimental.pallas import tpu_sc as plsc

NC = 2
NS = 16
NW = NC * NS

CHUNK = 128
NF = 32
BLK = 32768


def _dense_body(wt_ref, ht_ref, wb_ref, hb_ref, out_ref):
    prod = wt_ref[...] * ht_ref[...]
    out_ref[...] = jnp.sum(prod, axis=0) + wb_ref[...] + hb_ref[...]


def _gather_body(nodes_hbm, dsum_hbm, out_hbm, idx_v, val_v, sem):
    n_chunks = nodes_hbm.shape[0] // NW
    wid = lax.axis_index("s") * NC + lax.axis_index("c")
    base_row = wid * n_chunks

    pltpu.sync_copy(nodes_hbm.at[pl.ds(base_row, n_chunks)], idx_v)
    for k in range(n_chunks):
        pltpu.async_copy(dsum_hbm.at[idx_v.at[k]], val_v.at[k], sem)
    for k in range(n_chunks):
        pltpu.make_async_copy(dsum_hbm.at[idx_v.at[k]], val_v.at[k],
                              sem).wait()
    for k in range(n_chunks):
        pltpu.sync_copy(
            val_v.at[k],
            out_hbm.at[pl.ds((base_row + k) * CHUNK, CHUNK)])


def kernel(nodes, W, H, w_bias, h_bias):
    batch = nodes.shape[0]
    nn = W.shape[0]
    wt = jnp.transpose(W)
    ht = jnp.transpose(H)
    wb = jnp.reshape(w_bias, (nn,))
    hb = jnp.reshape(h_bias, (nn,))

    grid = (nn + BLK - 1) // BLK
    dsum = pl.pallas_call(
        _dense_body,
        grid=(grid,),
        in_specs=[
            pl.BlockSpec((NF, BLK), lambda i: (0, i)),
            pl.BlockSpec((NF, BLK), lambda i: (0, i)),
            pl.BlockSpec((BLK,), lambda i: (i,)),
            pl.BlockSpec((BLK,), lambda i: (i,)),
        ],
        out_specs=pl.BlockSpec((BLK,), lambda i: (i,)),
        out_shape=jax.ShapeDtypeStruct((nn,), jnp.float32),
        compiler_params=pltpu.CompilerParams(
            dimension_semantics=("arbitrary",)),
    )(wt, ht, wb, hb)

    nodes2d = jnp.reshape(nodes.astype(jnp.int32), (batch // CHUNK, CHUNK))
    mesh = plsc.VectorSubcoreMesh(core_axis_name="c", subcore_axis_name="s")
    n_chunks = nodes2d.shape[0] // NW
    run = pl.kernel(
        _gather_body,
        out_type=jax.ShapeDtypeStruct((batch,), jnp.float32),
        mesh=mesh,
        scratch_types=[
            pltpu.VMEM((n_chunks, CHUNK), jnp.int32),
            pltpu.VMEM((n_chunks, CHUNK), jnp.float32),
            pltpu.SemaphoreType.DMA,
        ],
    )
    return run(nodes2d, dsum)
